# baseline (speedup 1.0000x reference)
import jax
import jax.numpy as jnp
from jax import lax
from jax.experimental import pallas as pl
from jax.experimental.pallas import tpu as pltpu
from jax.experimental.pallas import tpu_sc as plsc

N = 50000
D = 128
EPS = 1e-5

_ET = 800000
_NC = 2
_NS = 16
_L = 16
_EPT = _ET // _NS
_SEG = 2000
_NSEG = _EPT // _SEG
_NP = 2
_R = 12544
_RPT = _R // _NS
_K = 64
_CAP = _SEG + 176
_NRANGE = _NC * _NP
_LAST_BASE = 3 * _R + (_NS - 1) * _RPT
_LAST_ROWS = N - _LAST_BASE


def _sc_scatter_body(h, src, dst, ew, out,
                     acc, stg_d, stg_s, stg_w, cidx, csrc, cew,
                     fidx0, rows0, gsem0, ssem0,
                     fidx1, rows1, gsem1, ssem1):
    c = lax.axis_index("c")
    s = lax.axis_index("s")
    ebase = s * _EPT

    def prep_gather(t, fi, rows_q, gsem_q):
        boff = t * _K
        for k in range(_K // _L):
            fi[pl.ds(k * _L, _L)] = cidx[pl.ds(boff + k * _L, _L)]
        pltpu.async_copy(h.at[csrc.at[pl.ds(boff, _K)]], rows_q, gsem_q)

    def wait_gather(t, rows_q, gsem_q):
        boff = t * _K
        pltpu.make_async_copy(h.at[csrc.at[pl.ds(boff, _K)]], rows_q,
                              gsem_q).wait()

    def scale(t, rows_q):
        boff = t * _K

        def scale_g(gg, carry4):
            w16 = cew[pl.ds(boff + gg * _L, _L)]
            for e in range(_L):
                wspl = jnp.take_along_axis(
                    w16, jnp.full((_L,), e, jnp.int32), axis=0)
                r = gg * _L + e
                for k in range(D // _L):
                    rows_q[r, pl.ds(k * _L, _L)] = (
                        rows_q[r, pl.ds(k * _L, _L)] * wspl)
            return carry4
        lax.fori_loop(0, _K // _L, scale_g, 0)

    def flush_batch(boff):
        t = boff // _K
        prep_gather(t, fidx0, rows0, gsem0)
        wait_gather(t, rows0, gsem0)
        scale(t, rows0)
        pltpu.async_copy(rows0, acc.at[fidx0], ssem0, add=True).wait()

    def do_pass(p, carry):
        rid = _NP * c + p
        lo = rid * _R

        plsc.subcore_barrier()

        def zrow(r, carry2):
            for k in range(D // _L):
                rows0[r, pl.ds(k * _L, _L)] = jnp.zeros((_L,), jnp.float32)
            return carry2
        lax.fori_loop(0, _K, zrow, 0)
        for i in range(_RPT // _K):
            pltpu.sync_copy(rows0, acc.at[pl.ds(s * _RPT + i * _K, _K)])
        pltpu.sync_copy(rows0.at[pl.ds(0, _RPT % _K)],
                        acc.at[pl.ds(s * _RPT + (_RPT // _K) * _K,
                                     _RPT % _K)])
        plsc.subcore_barrier()

        def do_seg(g, cur):
            off = ebase + g * _SEG
            pltpu.sync_copy(dst.at[pl.ds(off, _SEG)], stg_d)
            pltpu.sync_copy(src.at[pl.ds(off, _SEG)], stg_s)
            pltpu.sync_copy(ew.at[pl.ds(off, _SEG)], stg_w)

            def cvec(v, cur2):
                dv = stg_d[pl.ds(v * _L, _L)] - lo
                sv = stg_s[pl.ds(v * _L, _L)]
                wv = stg_w[pl.ds(v * _L, _L)]
                m = (dv >= 0) & (dv < _R)
                plsc.store_compressed(cidx.at[pl.ds(cur2, _L)], dv, mask=m)
                plsc.store_compressed(csrc.at[pl.ds(cur2, _L)], sv, mask=m)
                plsc.store_compressed(cew.at[pl.ds(cur2, _L)], wv, mask=m)
                return cur2 + jnp.sum(jnp.where(m, 1, 0))
            cur = lax.fori_loop(0, _SEG // _L, cvec, cur)

            nbf = cur // _K

            bufs = ((fidx0, rows0, gsem0, ssem0),
                    (fidx1, rows1, gsem1, ssem1))

            @pl.when(nbf > 0)
            def _():
                prep_gather(0, fidx0, rows0, gsem0)

            def pair(i, carry3):
                for j in range(2):
                    t = 2 * i + j
                    fi, rows_q, gsem_q, ssem_q = bufs[j]
                    fi2, rows_q2, gsem_q2, ssem_q2 = bufs[1 - j]

                    @pl.when(t < nbf)
                    def _():
                        @pl.when(t + 1 < nbf)
                        def _():
                            @pl.when(t >= 1)
                            def _():
                                pltpu.make_async_copy(
                                    rows_q2, acc.at[fi2], ssem_q2).wait()
                            prep_gather(t + 1, fi2, rows_q2, gsem_q2)
                        wait_gather(t, rows_q, gsem_q)
                        scale(t, rows_q)
                        pltpu.async_copy(rows_q, acc.at[fi], ssem_q,
                                         add=True)
                return carry3
            lax.fori_loop(0, (nbf + 1) // 2, pair, 0)

            for q in range(2):
                fi, rows_q, gsem_q, ssem_q = bufs[q]

                @pl.when(q < nbf)
                def _():
                    pltpu.make_async_copy(rows_q, acc.at[fi], ssem_q).wait()

            rem_off = nbf * _K
            for k in range(_K // _L):
                t0 = cidx[pl.ds(rem_off + k * _L, _L)]
                t1 = csrc[pl.ds(rem_off + k * _L, _L)]
                t2 = cew[pl.ds(rem_off + k * _L, _L)]
                cidx[pl.ds(k * _L, _L)] = t0
                csrc[pl.ds(k * _L, _L)] = t1
                cew[pl.ds(k * _L, _L)] = t2
            return cur - rem_off
        cur = lax.fori_loop(0, _NSEG, do_seg, 0)

        @pl.when(cur > 0)
        def _():
            for k in range(_K // _L):
                cidx[pl.ds(cur + k * _L, _L)] = jnp.zeros((_L,), jnp.int32)
                csrc[pl.ds(cur + k * _L, _L)] = jnp.zeros((_L,), jnp.int32)
                cew[pl.ds(cur + k * _L, _L)] = jnp.zeros((_L,), jnp.float32)
            flush_batch(0)

        plsc.subcore_barrier()

        rb = s * _RPT
        glo = lo + rb

        is_clip = (rid == 3) & (s == _NS - 1)

        @pl.when(jnp.logical_not(is_clip))
        def _():
            pltpu.sync_copy(acc.at[pl.ds(rb, _RPT)], out.at[pl.ds(glo, _RPT)])

        @pl.when(is_clip)
        def _():
            pltpu.sync_copy(acc.at[pl.ds(rb, _LAST_ROWS)],
                            out.at[pl.ds(glo, _LAST_ROWS)])
        return carry
    lax.fori_loop(0, _NP, do_pass, 0)


@jax.jit
def _sc_scatter(h, src, dst, ew):
    mesh = plsc.VectorSubcoreMesh(core_axis_name="c", subcore_axis_name="s")
    return pl.kernel(
        _sc_scatter_body,
        out_type=jax.ShapeDtypeStruct((N, D), jnp.float32),
        mesh=mesh,
        compiler_params=pltpu.CompilerParams(needs_layout_passes=False),
        scratch_types=[
            pltpu.VMEM_SHARED((_R, D), jnp.float32),
            pltpu.VMEM((_SEG,), jnp.int32),
            pltpu.VMEM((_SEG,), jnp.int32),
            pltpu.VMEM((_SEG,), jnp.float32),
            pltpu.VMEM((_CAP,), jnp.int32),
            pltpu.VMEM((_CAP,), jnp.int32),
            pltpu.VMEM((_CAP,), jnp.float32),
            pltpu.VMEM((_K,), jnp.int32),
            pltpu.VMEM((_K, D), jnp.float32),
            pltpu.SemaphoreType.DMA,
            pltpu.SemaphoreType.DMA,
            pltpu.VMEM((_K,), jnp.int32),
            pltpu.VMEM((_K, D), jnp.float32),
            pltpu.SemaphoreType.DMA,
            pltpu.SemaphoreType.DMA,
        ],
    )(h, src, dst, ew)


_BLK = 2000
_GRID = N // _BLK


def _mm_body(x_ref, w_ref, o_ref):
    o_ref[...] = jnp.dot(x_ref[...], w_ref[...],
                         preferred_element_type=jnp.float32)


def _matmul(x, W):
    return pl.pallas_call(
        _mm_body,
        grid=(_GRID,),
        in_specs=[pl.BlockSpec((_BLK, D), lambda i: (i, 0)),
                  pl.BlockSpec((D, D), lambda i: (0, 0))],
        out_specs=pl.BlockSpec((_BLK, D), lambda i: (i, 0)),
        out_shape=jax.ShapeDtypeStruct((N, D), jnp.float32),
    )(x, W)


def _stats_body(x_ref, cs_ref, cq_ref):
    i = pl.program_id(0)

    @pl.when(i == 0)
    def _():
        cs_ref[...] = jnp.zeros_like(cs_ref)
        cq_ref[...] = jnp.zeros_like(cq_ref)

    blk = x_ref[...]
    cs_ref[...] += jnp.sum(blk, axis=0, keepdims=True)
    cq_ref[...] += jnp.sum(blk * blk, axis=0, keepdims=True)


def _stats(agg):
    return pl.pallas_call(
        _stats_body,
        grid=(_GRID,),
        in_specs=[pl.BlockSpec((_BLK, D), lambda i: (i, 0))],
        out_specs=[pl.BlockSpec((1, D), lambda i: (0, 0)),
                   pl.BlockSpec((1, D), lambda i: (0, 0))],
        out_shape=[jax.ShapeDtypeStruct((1, D), jnp.float32),
                   jax.ShapeDtypeStruct((1, D), jnp.float32)],
    )(agg)


def _affine_from_stats(cs_ref, cq_ref, b_ref, lnw_ref, lnb_ref):
    c = 4.0 * b_ref[...]
    cs = cs_ref[...]
    cq = cq_ref[...]
    nd = float(N * D)
    mu = (jnp.sum(cs) + N * jnp.sum(c)) / nd
    e2 = (jnp.sum(cq) + 2.0 * jnp.sum(c * cs) + N * jnp.sum(c * c)) / nd
    sigma = jnp.sqrt(e2 - mu * mu + EPS)
    alpha = lnw_ref[...] / sigma
    beta = (c - mu) * alpha + lnb_ref[...]
    return alpha, beta


def _ln_mm_body(agg_ref, cs_ref, cq_ref, b_ref, lnw_ref, lnb_ref, w_ref,
                o_ref):
    alpha, beta = _affine_from_stats(cs_ref, cq_ref, b_ref, lnw_ref, lnb_ref)
    h = agg_ref[...] * alpha + beta
    o_ref[...] = jnp.dot(h, w_ref[...], preferred_element_type=jnp.float32)


def _ln_matmul(agg, cs, cq, b, lnw, lnb, W):
    vec = pl.BlockSpec((1, D), lambda i: (0, 0))
    return pl.pallas_call(
        _ln_mm_body,
        grid=(_GRID,),
        in_specs=[pl.BlockSpec((_BLK, D), lambda i: (i, 0)),
                  vec, vec, vec, vec, vec,
                  pl.BlockSpec((D, D), lambda i: (0, 0))],
        out_specs=pl.BlockSpec((_BLK, D), lambda i: (i, 0)),
        out_shape=jax.ShapeDtypeStruct((N, D), jnp.float32),
    )(agg, cs, cq, b.reshape(1, D), lnw.reshape(1, D), lnb.reshape(1, D), W)


def _ln_final_body(agg_ref, cs_ref, cq_ref, b_ref, lnw_ref, lnb_ref, o_ref):
    alpha, beta = _affine_from_stats(cs_ref, cq_ref, b_ref, lnw_ref, lnb_ref)
    o_ref[...] = agg_ref[...] * alpha + beta


def _ln_final(agg, cs, cq, b, lnw, lnb):
    vec = pl.BlockSpec((1, D), lambda i: (0, 0))
    return pl.pallas_call(
        _ln_final_body,
        grid=(_GRID,),
        in_specs=[pl.BlockSpec((_BLK, D), lambda i: (i, 0)),
                  vec, vec, vec, vec, vec],
        out_specs=pl.BlockSpec((_BLK, D), lambda i: (i, 0)),
        out_shape=jax.ShapeDtypeStruct((N, D), jnp.float32),
    )(agg, cs, cq, b.reshape(1, D), lnw.reshape(1, D), lnb.reshape(1, D))


def kernel(x, ei_forward, ei_onset, ei_sustain, ei_rest,
           ew_forward, ew_onset, ew_sustain, ew_rest,
           W1, b1, ln1_w, ln1_b, W2, b2, ln2_w, ln2_b):
    src = jnp.concatenate([ei_forward[0], ei_onset[0], ei_sustain[0],
                           ei_rest[0]])
    dst = jnp.concatenate([ei_forward[1], ei_onset[1], ei_sustain[1],
                           ei_rest[1]])
    ew = jnp.concatenate([ew_forward, ew_onset, ew_sustain, ew_rest])

    h1 = _matmul(x, W1)
    agg1 = _sc_scatter(h1, src, dst, ew)
    cs1, cq1 = _stats(agg1)
    h2 = _ln_matmul(agg1, cs1, cq1, b1, ln1_w, ln1_b, W2)
    agg2 = _sc_scatter(h2, src, dst, ew)
    cs2, cq2 = _stats(agg2)
    return _ln_final(agg2, cs2, cq2, b2, ln2_w, ln2_b)


# speedup vs baseline: 1.0212x; 1.0212x over previous
import jax
import jax.numpy as jnp
from jax import lax
from jax.experimental import pallas as pl
from jax.experimental.pallas import tpu as pltpu
from jax.experimental.pallas import tpu_sc as plsc

N = 50000
D = 128
EPS = 1e-5

_ET = 800000
_NC = 2
_NS = 16
_L = 16
_EPT = _ET // _NS
_SEG = 2000
_NSEG = _EPT // _SEG
_NP = 2
_R = 12544
_RPT = _R // _NS
_K = 64
_CAP = _SEG + 176
_NRANGE = _NC * _NP
_LAST_BASE = 3 * _R + (_NS - 1) * _RPT
_LAST_ROWS = N - _LAST_BASE


def _sc_scatter_body(h, src, dst, ew, out,
                     acc, stg_d, stg_s, stg_w, cidx, csrc, cew,
                     fidx0, rows0, gsem0, ssem0,
                     fidx1, rows1, gsem1, ssem1):
    c = lax.axis_index("c")
    s = lax.axis_index("s")
    ebase = s * _EPT

    def prep_gather(t, fi, rows_q, gsem_q):
        boff = t * _K
        for k in range(_K // _L):
            fi[pl.ds(k * _L, _L)] = cidx[pl.ds(boff + k * _L, _L)]
        pltpu.async_copy(h.at[csrc.at[pl.ds(boff, _K)]], rows_q, gsem_q)

    def wait_gather(t, rows_q, gsem_q):
        boff = t * _K
        pltpu.make_async_copy(h.at[csrc.at[pl.ds(boff, _K)]], rows_q,
                              gsem_q).wait()

    def scale(t, rows_q):
        boff = t * _K

        def scale_g(gg, carry4):
            w16 = cew[pl.ds(boff + gg * _L, _L)]
            for e in range(_L):
                wspl = jnp.take_along_axis(
                    w16, jnp.full((_L,), e, jnp.int32), axis=0)
                r = gg * _L + e
                for k in range(D // _L):
                    rows_q[r, pl.ds(k * _L, _L)] = (
                        rows_q[r, pl.ds(k * _L, _L)] * wspl)
            return carry4
        lax.fori_loop(0, _K // _L, scale_g, 0)

    def flush_batch(boff):
        t = boff // _K
        prep_gather(t, fidx0, rows0, gsem0)
        wait_gather(t, rows0, gsem0)
        scale(t, rows0)
        pltpu.async_copy(rows0, acc.at[fidx0], ssem0, add=True).wait()

    def do_pass(p, carry):
        rid = _NP * c + p
        lo = rid * _R

        plsc.subcore_barrier()

        def zrow(r, carry2):
            for k in range(D // _L):
                rows0[r, pl.ds(k * _L, _L)] = jnp.zeros((_L,), jnp.float32)
            return carry2
        lax.fori_loop(0, _K, zrow, 0)
        for i in range(_RPT // _K):
            pltpu.sync_copy(rows0, acc.at[pl.ds(s * _RPT + i * _K, _K)])
        pltpu.sync_copy(rows0.at[pl.ds(0, _RPT % _K)],
                        acc.at[pl.ds(s * _RPT + (_RPT // _K) * _K,
                                     _RPT % _K)])
        plsc.subcore_barrier()

        def do_seg(g, carry_seg):
            cur, out0, out1 = carry_seg
            off = ebase + g * _SEG
            pltpu.sync_copy(dst.at[pl.ds(off, _SEG)], stg_d)
            pltpu.sync_copy(src.at[pl.ds(off, _SEG)], stg_s)
            pltpu.sync_copy(ew.at[pl.ds(off, _SEG)], stg_w)

            def cvec(v, cur2):
                dv = stg_d[pl.ds(v * _L, _L)] - lo
                sv = stg_s[pl.ds(v * _L, _L)]
                wv = stg_w[pl.ds(v * _L, _L)]
                m = (dv >= 0) & (dv < _R)
                plsc.store_compressed(cidx.at[pl.ds(cur2, _L)], dv, mask=m)
                plsc.store_compressed(csrc.at[pl.ds(cur2, _L)], sv, mask=m)
                plsc.store_compressed(cew.at[pl.ds(cur2, _L)], wv, mask=m)
                return cur2 + jnp.sum(jnp.where(m, 1, 0))
            cur = lax.fori_loop(0, _SEG // _L, cvec, cur)

            nbf = cur // _K

            bufs = ((fidx0, rows0, gsem0, ssem0),
                    (fidx1, rows1, gsem1, ssem1))

            @pl.when(nbf > 0)
            def _():
                @pl.when(out0 > 0)
                def _():
                    pltpu.make_async_copy(rows0, acc.at[fidx0],
                                          ssem0).wait()
                prep_gather(0, fidx0, rows0, gsem0)

            def pair(i, carry3):
                for j in range(2):
                    t = 2 * i + j
                    fi, rows_q, gsem_q, ssem_q = bufs[j]
                    fi2, rows_q2, gsem_q2, ssem_q2 = bufs[1 - j]

                    @pl.when(t < nbf)
                    def _():
                        @pl.when(t + 1 < nbf)
                        def _():
                            @pl.when((t >= 1) | ((t == 0) & (out1 > 0)))
                            def _():
                                pltpu.make_async_copy(
                                    rows_q2, acc.at[fi2], ssem_q2).wait()
                            prep_gather(t + 1, fi2, rows_q2, gsem_q2)
                        wait_gather(t, rows_q, gsem_q)
                        scale(t, rows_q)
                        pltpu.async_copy(rows_q, acc.at[fi], ssem_q,
                                         add=True)
                return carry3
            lax.fori_loop(0, (nbf + 1) // 2, pair, 0)

            n_out0 = jnp.where(nbf >= 1, 1, out0)
            n_out1 = jnp.where(nbf >= 2, 1, out1)

            rem_off = nbf * _K
            for k in range(_K // _L):
                t0 = cidx[pl.ds(rem_off + k * _L, _L)]
                t1 = csrc[pl.ds(rem_off + k * _L, _L)]
                t2 = cew[pl.ds(rem_off + k * _L, _L)]
                cidx[pl.ds(k * _L, _L)] = t0
                csrc[pl.ds(k * _L, _L)] = t1
                cew[pl.ds(k * _L, _L)] = t2
            return (cur - rem_off, n_out0, n_out1)
        cur, out0, out1 = lax.fori_loop(0, _NSEG, do_seg, (0, 0, 0))

        @pl.when(out0 > 0)
        def _():
            pltpu.make_async_copy(rows0, acc.at[fidx0], ssem0).wait()

        @pl.when(out1 > 0)
        def _():
            pltpu.make_async_copy(rows1, acc.at[fidx1], ssem1).wait()

        @pl.when(cur > 0)
        def _():
            for k in range(_K // _L):
                cidx[pl.ds(cur + k * _L, _L)] = jnp.zeros((_L,), jnp.int32)
                csrc[pl.ds(cur + k * _L, _L)] = jnp.zeros((_L,), jnp.int32)
                cew[pl.ds(cur + k * _L, _L)] = jnp.zeros((_L,), jnp.float32)
            flush_batch(0)

        plsc.subcore_barrier()

        rb = s * _RPT
        glo = lo + rb

        is_clip = (rid == 3) & (s == _NS - 1)

        @pl.when(jnp.logical_not(is_clip))
        def _():
            pltpu.sync_copy(acc.at[pl.ds(rb, _RPT)], out.at[pl.ds(glo, _RPT)])

        @pl.when(is_clip)
        def _():
            pltpu.sync_copy(acc.at[pl.ds(rb, _LAST_ROWS)],
                            out.at[pl.ds(glo, _LAST_ROWS)])
        return carry
    lax.fori_loop(0, _NP, do_pass, 0)


@jax.jit
def _sc_scatter(h, src, dst, ew):
    mesh = plsc.VectorSubcoreMesh(core_axis_name="c", subcore_axis_name="s")
    return pl.kernel(
        _sc_scatter_body,
        out_type=jax.ShapeDtypeStruct((N, D), jnp.float32),
        mesh=mesh,
        compiler_params=pltpu.CompilerParams(needs_layout_passes=False),
        scratch_types=[
            pltpu.VMEM_SHARED((_R, D), jnp.float32),
            pltpu.VMEM((_SEG,), jnp.int32),
            pltpu.VMEM((_SEG,), jnp.int32),
            pltpu.VMEM((_SEG,), jnp.float32),
            pltpu.VMEM((_CAP,), jnp.int32),
            pltpu.VMEM((_CAP,), jnp.int32),
            pltpu.VMEM((_CAP,), jnp.float32),
            pltpu.VMEM((_K,), jnp.int32),
            pltpu.VMEM((_K, D), jnp.float32),
            pltpu.SemaphoreType.DMA,
            pltpu.SemaphoreType.DMA,
            pltpu.VMEM((_K,), jnp.int32),
            pltpu.VMEM((_K, D), jnp.float32),
            pltpu.SemaphoreType.DMA,
            pltpu.SemaphoreType.DMA,
        ],
    )(h, src, dst, ew)


_BLK = 2000
_GRID = N // _BLK


def _mm_body(x_ref, w_ref, o_ref):
    o_ref[...] = jnp.dot(x_ref[...], w_ref[...],
                         preferred_element_type=jnp.float32)


def _matmul(x, W):
    return pl.pallas_call(
        _mm_body,
        grid=(_GRID,),
        in_specs=[pl.BlockSpec((_BLK, D), lambda i: (i, 0)),
                  pl.BlockSpec((D, D), lambda i: (0, 0))],
        out_specs=pl.BlockSpec((_BLK, D), lambda i: (i, 0)),
        out_shape=jax.ShapeDtypeStruct((N, D), jnp.float32),
    )(x, W)


def _stats_body(x_ref, cs_ref, cq_ref):
    i = pl.program_id(0)

    @pl.when(i == 0)
    def _():
        cs_ref[...] = jnp.zeros_like(cs_ref)
        cq_ref[...] = jnp.zeros_like(cq_ref)

    blk = x_ref[...]
    cs_ref[...] += jnp.sum(blk, axis=0, keepdims=True)
    cq_ref[...] += jnp.sum(blk * blk, axis=0, keepdims=True)


def _stats(agg):
    return pl.pallas_call(
        _stats_body,
        grid=(_GRID,),
        in_specs=[pl.BlockSpec((_BLK, D), lambda i: (i, 0))],
        out_specs=[pl.BlockSpec((1, D), lambda i: (0, 0)),
                   pl.BlockSpec((1, D), lambda i: (0, 0))],
        out_shape=[jax.ShapeDtypeStruct((1, D), jnp.float32),
                   jax.ShapeDtypeStruct((1, D), jnp.float32)],
    )(agg)


def _affine_from_stats(cs_ref, cq_ref, b_ref, lnw_ref, lnb_ref):
    c = 4.0 * b_ref[...]
    cs = cs_ref[...]
    cq = cq_ref[...]
    nd = float(N * D)
    mu = (jnp.sum(cs) + N * jnp.sum(c)) / nd
    e2 = (jnp.sum(cq) + 2.0 * jnp.sum(c * cs) + N * jnp.sum(c * c)) / nd
    sigma = jnp.sqrt(e2 - mu * mu + EPS)
    alpha = lnw_ref[...] / sigma
    beta = (c - mu) * alpha + lnb_ref[...]
    return alpha, beta


def _ln_mm_body(agg_ref, cs_ref, cq_ref, b_ref, lnw_ref, lnb_ref, w_ref,
                o_ref):
    alpha, beta = _affine_from_stats(cs_ref, cq_ref, b_ref, lnw_ref, lnb_ref)
    h = agg_ref[...] * alpha + beta
    o_ref[...] = jnp.dot(h, w_ref[...], preferred_element_type=jnp.float32)


def _ln_matmul(agg, cs, cq, b, lnw, lnb, W):
    vec = pl.BlockSpec((1, D), lambda i: (0, 0))
    return pl.pallas_call(
        _ln_mm_body,
        grid=(_GRID,),
        in_specs=[pl.BlockSpec((_BLK, D), lambda i: (i, 0)),
                  vec, vec, vec, vec, vec,
                  pl.BlockSpec((D, D), lambda i: (0, 0))],
        out_specs=pl.BlockSpec((_BLK, D), lambda i: (i, 0)),
        out_shape=jax.ShapeDtypeStruct((N, D), jnp.float32),
    )(agg, cs, cq, b.reshape(1, D), lnw.reshape(1, D), lnb.reshape(1, D), W)


def _ln_final_body(agg_ref, cs_ref, cq_ref, b_ref, lnw_ref, lnb_ref, o_ref):
    alpha, beta = _affine_from_stats(cs_ref, cq_ref, b_ref, lnw_ref, lnb_ref)
    o_ref[...] = agg_ref[...] * alpha + beta


def _ln_final(agg, cs, cq, b, lnw, lnb):
    vec = pl.BlockSpec((1, D), lambda i: (0, 0))
    return pl.pallas_call(
        _ln_final_body,
        grid=(_GRID,),
        in_specs=[pl.BlockSpec((_BLK, D), lambda i: (i, 0)),
                  vec, vec, vec, vec, vec],
        out_specs=pl.BlockSpec((_BLK, D), lambda i: (i, 0)),
        out_shape=jax.ShapeDtypeStruct((N, D), jnp.float32),
    )(agg, cs, cq, b.reshape(1, D), lnw.reshape(1, D), lnb.reshape(1, D))


def kernel(x, ei_forward, ei_onset, ei_sustain, ei_rest,
           ew_forward, ew_onset, ew_sustain, ew_rest,
           W1, b1, ln1_w, ln1_b, W2, b2, ln2_w, ln2_b):
    src = jnp.concatenate([ei_forward[0], ei_onset[0], ei_sustain[0],
                           ei_rest[0]])
    dst = jnp.concatenate([ei_forward[1], ei_onset[1], ei_sustain[1],
                           ei_rest[1]])
    ew = jnp.concatenate([ew_forward, ew_onset, ew_sustain, ew_rest])

    h1 = _matmul(x, W1)
    agg1 = _sc_scatter(h1, src, dst, ew)
    cs1, cq1 = _stats(agg1)
    h2 = _ln_matmul(agg1, cs1, cq1, b1, ln1_w, ln1_b, W2)
    agg2 = _sc_scatter(h2, src, dst, ew)
    cs2, cq2 = _stats(agg2)
    return _ln_final(agg2, cs2, cq2, b2, ln2_w, ln2_b)
